# baseline (speedup 1.0000x reference)
import jax
import jax.numpy as jnp
from jax import lax
from jax.experimental import pallas as pl
from jax.experimental.pallas import tpu as pltpu
from jax.experimental.pallas import tpu_sc as plsc

NC = 2
NS = 16
NW = NC * NS
CHUNK = 64
LANES = 16


def _emb_body(tok_hbm, tab_hbm, pos_hbm, out_hbm, idx_v, rows_v, pos_v, sem):
    wid = lax.axis_index("s") * NC + lax.axis_index("c")
    n_rows = tok_hbm.shape[0]
    seq = pos_hbm.shape[0]
    d = tab_hbm.shape[1]
    vpr = d // LANES
    per_w = n_rows // NW
    n_chunks = per_w // CHUNK
    base = wid * per_w
    pos_base = lax.rem(base, seq)

    def chunk_body(ci, carry):
        b0 = base + ci * CHUNK
        p0 = pos_base + ci * CHUNK
        pltpu.sync_copy(tok_hbm.at[pl.ds(b0, CHUNK)], idx_v)
        gather = pltpu.async_copy(tab_hbm.at[idx_v], rows_v, sem)
        pltpu.sync_copy(pos_hbm.at[pl.ds(p0, CHUNK)], pos_v)
        gather.wait()

        def row_body(r, c2):
            def vec_body(c, c3):
                off = c * LANES
                rows_v[r, pl.ds(off, LANES)] = (
                    rows_v[r, pl.ds(off, LANES)] + pos_v[r, pl.ds(off, LANES)]
                )
                return c3

            return lax.fori_loop(0, vpr, vec_body, c2)

        lax.fori_loop(0, CHUNK, row_body, 0)
        pltpu.sync_copy(rows_v, out_hbm.at[pl.ds(b0, CHUNK)])
        return carry

    lax.fori_loop(0, n_chunks, chunk_body, 0)


def kernel(tokens, token_table, pos_table):
    B, L = tokens.shape
    V, D = token_table.shape
    tok_flat = tokens.reshape(B * L).astype(jnp.int32)
    mesh = plsc.VectorSubcoreMesh(
        core_axis_name="c", subcore_axis_name="s", num_cores=NC, num_subcores=NS
    )
    out = pl.kernel(
        _emb_body,
        out_type=jax.ShapeDtypeStruct((B * L, D), jnp.float32),
        mesh=mesh,
        scratch_types=[
            pltpu.VMEM((CHUNK,), jnp.int32),
            pltpu.VMEM((CHUNK, D), jnp.float32),
            pltpu.VMEM((CHUNK, D), jnp.float32),
            pltpu.SemaphoreType.DMA,
        ],
    )(tok_flat, token_table, pos_table)
    return out.reshape(B, L, D)


# speedup vs baseline: 1.2596x; 1.2596x over previous
import jax
import jax.numpy as jnp
from jax import lax
from jax.experimental import pallas as pl
from jax.experimental.pallas import tpu as pltpu
from jax.experimental.pallas import tpu_sc as plsc

NC = 2
NS = 16
NW = NC * NS
CHUNK = 32
LANES = 16


def _emb_body(tok_hbm, tab_hbm, pos_hbm, out_hbm, idx_v, rows_v, pos_v,
              gsem, psem, ssem):
    wid = lax.axis_index("s") * NC + lax.axis_index("c")
    nb = tok_hbm.shape[0]
    seq = pos_hbm.shape[0]
    d = tab_hbm.shape[1]
    per_w = seq // NW
    nl = per_w // CHUNK
    nsteps = nl * nb
    wl0 = wid * per_w

    def start_gather(s):
        li = s // nb
        b = lax.rem(s, nb)
        slot = lax.rem(s, 2)
        l0 = wl0 + li * CHUNK
        pltpu.sync_copy(tok_hbm.at[b, pl.ds(l0, CHUNK)], idx_v.at[slot])
        pltpu.async_copy(tab_hbm.at[idx_v.at[slot]], rows_v.at[slot], gsem)

    def start_pos(li):
        pslot = lax.rem(li, 2)
        l0 = wl0 + li * CHUNK
        pltpu.async_copy(pos_hbm.at[pl.ds(l0, CHUNK)], pos_v.at[pslot], psem)

    def wait_gather(slot):
        pltpu.make_async_copy(
            tab_hbm.at[idx_v.at[slot]], rows_v.at[slot], gsem).wait()

    def wait_pos(pslot):
        pltpu.make_async_copy(
            pos_hbm.at[pl.ds(0, CHUNK)], pos_v.at[pslot], psem).wait()

    def wait_one_store(slot):
        pltpu.make_async_copy(
            rows_v.at[slot], out_hbm.at[0, pl.ds(0, CHUNK)], ssem).wait()

    start_pos(0)
    start_gather(0)

    def step(s, carry):
        li = s // nb
        b = lax.rem(s, nb)
        slot = lax.rem(s, 2)
        nslot = 1 - slot

        @pl.when(s + 1 < nsteps)
        def _():
            @pl.when(b == nb - 1)
            def _():
                start_pos(li + 1)

            @pl.when(s >= 1)
            def _():
                wait_one_store(nslot)

            start_gather(s + 1)

        wait_gather(slot)

        @pl.when(b == 0)
        def _():
            wait_pos(lax.rem(li, 2))

        pslot = lax.rem(li, 2)

        def row_body(r, c):
            for v in range(d // LANES):
                off = v * LANES
                rows_v[slot, r, pl.ds(off, LANES)] = (
                    rows_v[slot, r, pl.ds(off, LANES)]
                    + pos_v[pslot, r, pl.ds(off, LANES)]
                )
            return c

        lax.fori_loop(0, CHUNK, row_body, 0)

        l0 = wl0 + li * CHUNK
        pltpu.async_copy(rows_v.at[slot], out_hbm.at[b, pl.ds(l0, CHUNK)],
                         ssem)
        return carry

    lax.fori_loop(0, nsteps, step, 0)
    wait_one_store(0)
    wait_one_store(1)


def kernel(tokens, token_table, pos_table):
    B, L = tokens.shape
    V, D = token_table.shape
    tok = tokens.astype(jnp.int32)
    mesh = plsc.VectorSubcoreMesh(
        core_axis_name="c", subcore_axis_name="s", num_cores=NC,
        num_subcores=NS
    )
    out = pl.kernel(
        _emb_body,
        out_type=jax.ShapeDtypeStruct((B, L, D), jnp.float32),
        mesh=mesh,
        scratch_types=[
            pltpu.VMEM((2, CHUNK), jnp.int32),
            pltpu.VMEM((2, CHUNK, D), jnp.float32),
            pltpu.VMEM((2, CHUNK, D), jnp.float32),
            pltpu.SemaphoreType.DMA,
            pltpu.SemaphoreType.DMA,
            pltpu.SemaphoreType.DMA,
        ],
    )(tok, token_table, pos_table)
    return out


# speedup vs baseline: 1.3218x; 1.0494x over previous
import jax
import jax.numpy as jnp
from jax import lax
from jax.experimental import pallas as pl
from jax.experimental.pallas import tpu as pltpu
from jax.experimental.pallas import tpu_sc as plsc

NC = 2
NS = 16
NW = NC * NS
CHUNK = 32
LANES = 16


def _emb_body(tok_hbm, tab_hbm, pos_hbm, out_hbm, idx_v, rows_v, pos_v,
              gsem, psem, ssem):
    wid = lax.axis_index("s") * NC + lax.axis_index("c")
    nb = tok_hbm.shape[0]
    seq = pos_hbm.shape[0]
    d = tab_hbm.shape[1]
    per_w = seq // NW
    nl = per_w // CHUNK
    nsteps = nl * nb
    wl0 = wid * per_w

    pltpu.sync_copy(tok_hbm.at[:, pl.ds(wl0, per_w)], idx_v)

    def start_gather(s):
        li = s // nb
        b = lax.rem(s, nb)
        slot = lax.rem(s, 2)
        pltpu.async_copy(
            tab_hbm.at[idx_v.at[b, pl.ds(li * CHUNK, CHUNK)]],
            rows_v.at[slot], gsem)

    def start_pos(li):
        pslot = lax.rem(li, 2)
        l0 = wl0 + li * CHUNK
        pltpu.async_copy(pos_hbm.at[pl.ds(l0, CHUNK)], pos_v.at[pslot], psem)

    def wait_gather(slot):
        pltpu.make_async_copy(
            tab_hbm.at[idx_v.at[0, pl.ds(0, CHUNK)]], rows_v.at[slot],
            gsem).wait()

    def wait_pos(pslot):
        pltpu.make_async_copy(
            pos_hbm.at[pl.ds(0, CHUNK)], pos_v.at[pslot], psem).wait()

    def wait_one_store(slot):
        pltpu.make_async_copy(
            rows_v.at[slot], out_hbm.at[0, pl.ds(0, CHUNK)], ssem).wait()

    start_pos(0)
    start_gather(0)

    def step(s, carry):
        li = s // nb
        b = lax.rem(s, nb)
        slot = lax.rem(s, 2)
        nslot = 1 - slot

        @pl.when(s + 1 < nsteps)
        def _():
            @pl.when(b == nb - 1)
            def _():
                start_pos(li + 1)

            @pl.when(s >= 1)
            def _():
                wait_one_store(nslot)

            start_gather(s + 1)

        wait_gather(slot)

        @pl.when(b == 0)
        def _():
            wait_pos(lax.rem(li, 2))

        pslot = lax.rem(li, 2)

        def row_body(r, c):
            for v in range(d // LANES):
                off = v * LANES
                rows_v[slot, r, pl.ds(off, LANES)] = (
                    rows_v[slot, r, pl.ds(off, LANES)]
                    + pos_v[pslot, r, pl.ds(off, LANES)]
                )
            return c

        lax.fori_loop(0, CHUNK, row_body, 0)

        l0 = wl0 + li * CHUNK
        pltpu.async_copy(rows_v.at[slot], out_hbm.at[b, pl.ds(l0, CHUNK)],
                         ssem)
        return carry

    lax.fori_loop(0, nsteps, step, 0)
    wait_one_store(0)
    wait_one_store(1)


def kernel(tokens, token_table, pos_table):
    B, L = tokens.shape
    V, D = token_table.shape
    tok = tokens.astype(jnp.int32)
    mesh = plsc.VectorSubcoreMesh(
        core_axis_name="c", subcore_axis_name="s", num_cores=NC,
        num_subcores=NS
    )
    out = pl.kernel(
        _emb_body,
        out_type=jax.ShapeDtypeStruct((B, L, D), jnp.float32),
        mesh=mesh,
        scratch_types=[
            pltpu.VMEM((B, L // NW), jnp.int32),
            pltpu.VMEM((2, CHUNK, D), jnp.float32),
            pltpu.VMEM((2, CHUNK, D), jnp.float32),
            pltpu.SemaphoreType.DMA,
            pltpu.SemaphoreType.DMA,
            pltpu.SemaphoreType.DMA,
        ],
    )(tok, token_table, pos_table)
    return out


# speedup vs baseline: 1.3420x; 1.0153x over previous
import jax
import jax.numpy as jnp
from jax import lax
from jax.experimental import pallas as pl
from jax.experimental.pallas import tpu as pltpu
from jax.experimental.pallas import tpu_sc as plsc

NC = 2
NS = 16
NW = NC * NS
CHUNK = 32
LANES = 16


def _emb_body(tok_hbm, tab_hbm, pos_hbm, out_hbm, idx_v, rows_v, dst_v,
              pos_v, gsem, ssem):
    wid = lax.axis_index("s") * NC + lax.axis_index("c")
    nb = tok_hbm.shape[0]
    seq = pos_hbm.shape[0]
    d = tab_hbm.shape[1]
    per_w = seq // NW
    nl = per_w // CHUNK
    nsteps = nl * nb
    wl0 = wid * per_w

    pltpu.sync_copy(tok_hbm.at[:, pl.ds(wl0, per_w)], idx_v)

    def start_gather(s):
        li = s // nb
        b = lax.rem(s, nb)
        slot = lax.rem(s, 2)
        pltpu.async_copy(
            tab_hbm.at[idx_v.at[b, pl.ds(li * CHUNK, CHUNK)]],
            rows_v.at[slot], gsem)

    def wait_gather(slot):
        pltpu.make_async_copy(
            tab_hbm.at[idx_v.at[0, pl.ds(0, CHUNK)]], rows_v.at[slot],
            gsem).wait()

    def wait_one_store(slot):
        pltpu.make_async_copy(
            dst_v.at[slot], out_hbm.at[0, pl.ds(0, CHUNK)], ssem).wait()

    start_gather(0)
    pltpu.sync_copy(pos_hbm.at[pl.ds(wl0, CHUNK)], pos_v)

    def step(s, carry):
        li = s // nb
        b = lax.rem(s, nb)
        slot = lax.rem(s, 2)

        @pl.when(s + 1 < nsteps)
        def _():
            start_gather(s + 1)

        @pl.when(s >= 2)
        def _():
            wait_one_store(slot)

        wait_gather(slot)

        @pl.when(jnp.logical_and(b == 0, s > 0))
        def _():
            pltpu.sync_copy(
                pos_hbm.at[pl.ds(wl0 + li * CHUNK, CHUNK)], pos_v)

        def row_body(r, c):
            for v in range(d // LANES):
                off = v * LANES
                dst_v[slot, r, pl.ds(off, LANES)] = (
                    rows_v[slot, r, pl.ds(off, LANES)]
                    + pos_v[r, pl.ds(off, LANES)]
                )
            return c

        lax.fori_loop(0, CHUNK, row_body, 0)

        l0 = wl0 + li * CHUNK
        pltpu.async_copy(dst_v.at[slot], out_hbm.at[b, pl.ds(l0, CHUNK)],
                         ssem)
        return carry

    lax.fori_loop(0, nsteps, step, 0)
    wait_one_store(0)
    wait_one_store(1)


def kernel(tokens, token_table, pos_table):
    B, L = tokens.shape
    V, D = token_table.shape
    tok = tokens.astype(jnp.int32)
    mesh = plsc.VectorSubcoreMesh(
        core_axis_name="c", subcore_axis_name="s", num_cores=NC,
        num_subcores=NS
    )
    out = pl.kernel(
        _emb_body,
        out_type=jax.ShapeDtypeStruct((B, L, D), jnp.float32),
        mesh=mesh,
        scratch_types=[
            pltpu.VMEM((B, L // NW), jnp.int32),
            pltpu.VMEM((2, CHUNK, D), jnp.float32),
            pltpu.VMEM((2, CHUNK, D), jnp.float32),
            pltpu.VMEM((CHUNK, D), jnp.float32),
            pltpu.SemaphoreType.DMA,
            pltpu.SemaphoreType.DMA,
        ],
    )(tok, token_table, pos_table)
    return out


# speedup vs baseline: 3.2654x; 2.4332x over previous
import jax
import jax.numpy as jnp
from jax import lax
from jax.experimental import pallas as pl
from jax.experimental.pallas import tpu as pltpu
from jax.experimental.pallas import tpu_sc as plsc

NC = 2
NS = 16
NW = NC * NS
CHUNK = 32
LANES = 16


def _emb_body(tok_hbm, tab_hbm, pos_hbm, out_hbm, idx_v, rows_v, dst_v,
              pos_v, gsem, ssem):
    wid = lax.axis_index("s") * NC + lax.axis_index("c")
    nb = tok_hbm.shape[0]
    seq = pos_hbm.shape[0]
    d = tab_hbm.shape[1]
    per_w = seq // NW
    nl = per_w // CHUNK
    nsteps = nl * nb
    wl0 = wid * per_w

    pltpu.sync_copy(tok_hbm.at[:, pl.ds(wl0, per_w)], idx_v)

    def start_gather(s):
        li = s // nb
        b = lax.rem(s, nb)
        slot = lax.rem(s, 2)
        pltpu.async_copy(
            tab_hbm.at[idx_v.at[b, pl.ds(li * CHUNK, CHUNK)]],
            rows_v.at[slot], gsem)

    def wait_gather(slot):
        pltpu.make_async_copy(
            tab_hbm.at[idx_v.at[0, pl.ds(0, CHUNK)]], rows_v.at[slot],
            gsem).wait()

    def wait_one_store(slot):
        pltpu.make_async_copy(
            dst_v.at[slot], out_hbm.at[0, pl.ds(0, CHUNK)], ssem).wait()

    start_gather(0)
    pltpu.sync_copy(pos_hbm.at[pl.ds(wl0, CHUNK)], pos_v)

    def step(s, carry):
        li = s // nb
        b = lax.rem(s, nb)
        slot = lax.rem(s, 2)

        @pl.when(s + 1 < nsteps)
        def _():
            start_gather(s + 1)

        @pl.when(s >= 2)
        def _():
            wait_one_store(slot)

        wait_gather(slot)

        @pl.when(jnp.logical_and(b == 0, s > 0))
        def _():
            pltpu.sync_copy(
                pos_hbm.at[pl.ds(wl0 + li * CHUNK, CHUNK)], pos_v)

        @plsc.parallel_loop(0, CHUNK, unroll=2)
        def _row(r):
            for g in range(0, d // LANES, 8):
                a = [rows_v[slot, r, pl.ds((g + i) * LANES, LANES)]
                     for i in range(8)]
                p = [pos_v[r, pl.ds((g + i) * LANES, LANES)]
                     for i in range(8)]
                for i in range(8):
                    dst_v[slot, r, pl.ds((g + i) * LANES, LANES)] = a[i] + p[i]

        l0 = wl0 + li * CHUNK
        pltpu.async_copy(dst_v.at[slot], out_hbm.at[b, pl.ds(l0, CHUNK)],
                         ssem)
        return carry

    lax.fori_loop(0, nsteps, step, 0)
    wait_one_store(0)
    wait_one_store(1)


def kernel(tokens, token_table, pos_table):
    B, L = tokens.shape
    V, D = token_table.shape
    tok = tokens.astype(jnp.int32)
    mesh = plsc.VectorSubcoreMesh(
        core_axis_name="c", subcore_axis_name="s", num_cores=NC,
        num_subcores=NS
    )
    out = pl.kernel(
        _emb_body,
        out_type=jax.ShapeDtypeStruct((B, L, D), jnp.float32),
        mesh=mesh,
        scratch_types=[
            pltpu.VMEM((B, L // NW), jnp.int32),
            pltpu.VMEM((2, CHUNK, D), jnp.float32),
            pltpu.VMEM((2, CHUNK, D), jnp.float32),
            pltpu.VMEM((CHUNK, D), jnp.float32),
            pltpu.SemaphoreType.DMA,
            pltpu.SemaphoreType.DMA,
        ],
    )(tok, token_table, pos_table)
    return out
